# baseline (speedup 1.0000x reference)
import functools
import math

import jax
import jax.numpy as jnp
from jax import lax
from jax.experimental import pallas as pl
from jax.experimental.pallas import tpu as pltpu
from jax.experimental.pallas import tpu_sc as plsc

D_MODEL = 64
SCALE = math.sqrt(D_MODEL)
NUM_CORES = 2
NUM_SUBCORES = 16
NW = NUM_CORES * NUM_SUBCORES
CHUNK = 128


@functools.lru_cache(maxsize=None)
def _make_embed(n_total: int, vocab: int):
    assert n_total % (NW * CHUNK) == 0
    n_per_w = n_total // NW
    n_chunks = n_per_w // CHUNK
    mesh = plsc.VectorSubcoreMesh(core_axis_name="c", subcore_axis_name="s")

    @functools.partial(
        pl.kernel,
        mesh=mesh,
        compiler_params=pltpu.CompilerParams(use_tc_tiling_on_sc=False),
        out_type=jax.ShapeDtypeStruct((n_total, D_MODEL), jnp.float32),
        scratch_types=[
            pltpu.VMEM((CHUNK,), jnp.int32),
            pltpu.VMEM((CHUNK, D_MODEL), jnp.float32),
            pltpu.SemaphoreType.DMA,
        ],
    )
    def embed(idx_hbm, lut_hbm, out_hbm, idx_v, rows_v, sem):
        wid = lax.axis_index("s") * NUM_CORES + lax.axis_index("c")
        base = wid * n_per_w

        def chunk_body(c, carry):
            off = base + c * CHUNK
            pltpu.sync_copy(idx_hbm.at[pl.ds(off, CHUNK)], idx_v)
            pltpu.async_copy(lut_hbm.at[idx_v], rows_v, sem).wait()

            def row_body(r, carry2):
                for j in range(D_MODEL // 16):
                    sl = pl.ds(j * 16, 16)
                    rows_v[r, sl] = rows_v[r, sl] * SCALE
                return carry2

            lax.fori_loop(0, CHUNK, row_body, 0, unroll=4)
            pltpu.sync_copy(rows_v, out_hbm.at[pl.ds(off, CHUNK)])
            return carry

        lax.fori_loop(0, n_chunks, chunk_body, 0)

    return embed


def kernel(x, lut):
    b, s = x.shape
    n = b * s
    idx = x.reshape(n).astype(jnp.int32)
    out = _make_embed(n, lut.shape[0])(idx, lut)
    return out.reshape(b, s, D_MODEL)


# speedup vs baseline: 1.1140x; 1.1140x over previous
import functools
import math

import jax
import jax.numpy as jnp
from jax import lax
from jax.experimental import pallas as pl
from jax.experimental.pallas import tpu as pltpu
from jax.experimental.pallas import tpu_sc as plsc

D_MODEL = 64
SCALE = math.sqrt(D_MODEL)
NUM_CORES = 2
NUM_SUBCORES = 16
NW = NUM_CORES * NUM_SUBCORES
CHUNK = 128
NBUF = 4


@functools.lru_cache(maxsize=None)
def _make_embed(n_total: int, vocab: int):
    assert n_total % (NW * CHUNK * NBUF) == 0
    n_per_w = n_total // NW
    n_chunks = n_per_w // CHUNK
    n_groups = n_chunks // NBUF
    mesh = plsc.VectorSubcoreMesh(core_axis_name="c", subcore_axis_name="s")

    @functools.partial(
        pl.kernel,
        mesh=mesh,
        compiler_params=pltpu.CompilerParams(use_tc_tiling_on_sc=False),
        out_type=jax.ShapeDtypeStruct((n_total, D_MODEL), jnp.float32),
        scratch_types=[
            pltpu.VMEM((n_chunks, CHUNK), jnp.int32),
            pltpu.VMEM((NBUF, CHUNK, D_MODEL), jnp.float32),
            pltpu.VMEM((NBUF, CHUNK, D_MODEL), jnp.float32),
            pltpu.SemaphoreType.DMA((NBUF,)),
            pltpu.SemaphoreType.DMA((NBUF,)),
        ],
    )
    def embed(idx_hbm, lut_hbm, out_hbm, idx_v, gbuf, wbuf, gsem, wsem):
        wid = lax.axis_index("s") * NUM_CORES + lax.axis_index("c")
        base = wid * n_per_w
        pltpu.sync_copy(idx_hbm.at[pl.ds(wid * n_chunks, n_chunks)], idx_v)

        for b in range(NBUF):
            pltpu.async_copy(lut_hbm.at[idx_v.at[b]], gbuf.at[b], gsem.at[b])

        def group_body(cc, carry):
            for b in range(NBUF):
                c = cc * NBUF + b
                pltpu.make_async_copy(
                    lut_hbm.at[idx_v.at[0]], gbuf.at[b], gsem.at[b]
                ).wait()

                @pl.when(cc > 0)
                def _wait_wb():
                    pltpu.make_async_copy(
                        wbuf.at[b], out_hbm.at[pl.ds(0, CHUNK)], wsem.at[b]
                    ).wait()

                def row_body(r, carry2):
                    for j in range(D_MODEL // 16):
                        sl = pl.ds(j * 16, 16)
                        wbuf[b, r, sl] = gbuf[b, r, sl] * SCALE
                    return carry2

                lax.fori_loop(0, CHUNK, row_body, 0, unroll=4)

                pltpu.async_copy(
                    wbuf.at[b],
                    out_hbm.at[pl.ds(base + c * CHUNK, CHUNK)],
                    wsem.at[b],
                )

                @pl.when(cc < n_groups - 1)
                def _next_gather():
                    pltpu.async_copy(
                        lut_hbm.at[idx_v.at[c + NBUF]], gbuf.at[b], gsem.at[b]
                    )

            return carry

        lax.fori_loop(0, n_groups, group_body, 0)

        for b in range(NBUF):
            pltpu.make_async_copy(
                wbuf.at[b], out_hbm.at[pl.ds(0, CHUNK)], wsem.at[b]
            ).wait()

    return embed


def kernel(x, lut):
    b, s = x.shape
    n = b * s
    idx = x.reshape(n // CHUNK, CHUNK).astype(jnp.int32)
    out = _make_embed(n, lut.shape[0])(idx, lut)
    return out.reshape(b, s, D_MODEL)
